# Initial kernel scaffold; baseline (speedup 1.0000x reference)
import jax
import jax.numpy as jnp
from jax.experimental import pallas as pl


def kernel(input_data, pos_data, word_table, pos_table):
    raise NotImplementedError("write your pallas kernel here")


# speedup vs baseline: 1.3091x; 1.3091x over previous
import functools

import jax
import jax.numpy as jnp
from jax import lax
from jax.experimental import pallas as pl
from jax.experimental.pallas import tpu as pltpu
from jax.experimental.pallas import tpu_sc as plsc

EMB = 64
B, L = 1024, 200
N_IDX = B * L
NC, NS = 2, 16
NW = NC * NS
CHUNK = 128
ROWS_PER_W = N_IDX // NW
CHUNKS_PER_W = ROWS_PER_W // CHUNK
G = 5
N_GROUPS = CHUNKS_PER_W // G

_mesh = plsc.VectorSubcoreMesh(
    core_axis_name="c", subcore_axis_name="s", num_cores=NC, num_subcores=NS
)


@functools.partial(
    pl.kernel,
    out_type=jax.ShapeDtypeStruct((N_IDX, EMB), jnp.float32),
    mesh=_mesh,
    compiler_params=pltpu.CompilerParams(use_tc_tiling_on_sc=False),
    scratch_types=[
        pltpu.VMEM((ROWS_PER_W,), jnp.int32),
        pltpu.VMEM((ROWS_PER_W,), jnp.int32),
        pltpu.VMEM((G, CHUNK, EMB), jnp.float32),
        pltpu.SemaphoreType.DMA,
        pltpu.SemaphoreType.DMA,
    ],
)
def _emb_lookup(widx_hbm, pidx_hbm, word_hbm, pos_hbm, out_hbm,
                widx_v, pidx_v, rows_v, gsem, osem):
    wid = lax.axis_index("s") * NC + lax.axis_index("c")
    out_base = wid * ROWS_PER_W
    pltpu.sync_copy(widx_hbm.at[pl.ds(out_base, ROWS_PER_W)], widx_v)
    pltpu.sync_copy(pidx_hbm.at[pl.ds(out_base, ROWS_PER_W)], pidx_v)

    def body(g, carry):
        j0 = g * G
        gathers = [
            pltpu.async_copy(
                word_hbm.at[widx_v.at[pl.ds((j0 + b) * CHUNK, CHUNK)]],
                rows_v.at[b], gsem)
            for b in range(G)
        ]
        for c in gathers:
            c.wait()
        adds = [
            pltpu.async_copy(
                pos_hbm.at[pidx_v.at[pl.ds((j0 + b) * CHUNK, CHUNK)]],
                rows_v.at[b], gsem, add=True)
            for b in range(G)
        ]
        for c in adds:
            c.wait()
        outs = [
            pltpu.async_copy(
                rows_v.at[b],
                out_hbm.at[pl.ds(out_base + (j0 + b) * CHUNK, CHUNK)],
                osem,
            )
            for b in range(G)
        ]
        for c in outs:
            c.wait()
        return carry

    lax.fori_loop(0, N_GROUPS, body, 0)


def kernel(input_data, pos_data, word_table, pos_table):
    widx = input_data.reshape(N_IDX).astype(jnp.int32)
    pidx = pos_data.reshape(N_IDX).astype(jnp.int32)
    out = _emb_lookup(widx, pidx, word_table, pos_table)
    return out.reshape(B, L, EMB)


# speedup vs baseline: 1.3548x; 1.0349x over previous
import functools

import jax
import jax.numpy as jnp
from jax import lax
from jax.experimental import pallas as pl
from jax.experimental.pallas import tpu as pltpu
from jax.experimental.pallas import tpu_sc as plsc

VOCAB = 1000000
MAX_LEN = 2048
EMB = 64
B, L = 1024, 200
N_IDX = B * L
NC, NS = 2, 16
NW = NC * NS
CHUNK = 128
ROWS_PER_W = N_IDX // NW
CHUNKS_PER_W = ROWS_PER_W // CHUNK
NBUF = 5
N_ITER = CHUNKS_PER_W // NBUF

_mesh = plsc.VectorSubcoreMesh(
    core_axis_name="c", subcore_axis_name="s", num_cores=NC, num_subcores=NS
)


@functools.partial(
    pl.kernel,
    out_type=jax.ShapeDtypeStruct((N_IDX, EMB), jnp.float32),
    mesh=_mesh,
    compiler_params=pltpu.CompilerParams(use_tc_tiling_on_sc=False),
    scratch_types=[
        pltpu.VMEM((ROWS_PER_W,), jnp.int32),
        pltpu.VMEM((ROWS_PER_W,), jnp.int32),
        pltpu.VMEM((NBUF, CHUNK, EMB), jnp.float32),
        pltpu.VMEM_SHARED((MAX_LEN, EMB), jnp.float32),
        pltpu.SemaphoreType.DMA,
        pltpu.SemaphoreType.DMA,
        pltpu.SemaphoreType.DMA,
    ],
)
def _emb_lookup(widx_hbm, pidx_hbm, word_hbm, pos_hbm, out_hbm,
                widx_v, pidx_v, rows_v, pos_sh, wsem, asem, osem):
    sid = lax.axis_index("s")
    wid = sid * NC + lax.axis_index("c")
    out_base = wid * ROWS_PER_W

    icp1 = pltpu.async_copy(widx_hbm.at[pl.ds(out_base, ROWS_PER_W)], widx_v, wsem)
    icp2 = pltpu.async_copy(pidx_hbm.at[pl.ds(out_base, ROWS_PER_W)], pidx_v, wsem)

    @pl.when(sid == 0)
    def _stage_pos_table():
        pltpu.sync_copy(pos_hbm, pos_sh)

    icp1.wait()
    icp2.wait()
    plsc.subcore_barrier()

    def word_cp(j, b):
        return pltpu.async_copy(
            word_hbm.at[widx_v.at[pl.ds(j * CHUNK, CHUNK)]], rows_v.at[b], wsem)

    def pos_cp(j, b):
        return pltpu.async_copy(
            pos_sh.at[pidx_v.at[pl.ds(j * CHUNK, CHUNK)]], rows_v.at[b], asem,
            add=True)

    def out_cp(j, b):
        return pltpu.make_async_copy(
            rows_v.at[b], out_hbm.at[pl.ds(out_base + j * CHUNK, CHUNK)], osem)

    def body(i, carry):
        j0 = i * NBUF

        @pl.when(i > 0)
        def _reclaim_buffers():
            for b in range(NBUF):
                out_cp(j0 - NBUF + b, b).wait()

        wcps = [word_cp(j0 + b, b) for b in range(NBUF)]
        acps = []
        for b in range(NBUF):
            wcps[b].wait()
            acps.append(pos_cp(j0 + b, b))
        for b in range(NBUF):
            acps[b].wait()
            out_cp(j0 + b, b).start()
        return carry

    lax.fori_loop(0, N_ITER, body, 0)
    for b in range(NBUF):
        out_cp((N_ITER - 1) * NBUF + b, b).wait()


def kernel(input_data, pos_data, word_table, pos_table):
    widx = input_data.reshape(N_IDX).astype(jnp.int32)
    pidx = pos_data.reshape(N_IDX).astype(jnp.int32)
    out = _emb_lookup(widx, pidx, word_table, pos_table)
    return out.reshape(B, L, EMB)


# speedup vs baseline: 1.4376x; 1.0611x over previous
import functools

import jax
import jax.numpy as jnp
from jax import lax
from jax.experimental import pallas as pl
from jax.experimental.pallas import tpu as pltpu
from jax.experimental.pallas import tpu_sc as plsc
import jax.experimental.layout as _layout

VOCAB = 1000000
MAX_LEN = 2048
EMB = 64
B, L = 1024, 200
N_IDX = B * L
NC, NS = 2, 16
NW = NC * NS
CHUNK = 128
ROWS_PER_W = N_IDX // NW
CHUNKS_PER_W = ROWS_PER_W // CHUNK
NBUF = 5
N_ITER = CHUNKS_PER_W // NBUF

_mesh = plsc.VectorSubcoreMesh(
    core_axis_name="c", subcore_axis_name="s", num_cores=NC, num_subcores=NS
)


@functools.partial(
    pl.kernel,
    out_type=jax.ShapeDtypeStruct((N_IDX, EMB), jnp.float32),
    mesh=_mesh,
    compiler_params=pltpu.CompilerParams(use_tc_tiling_on_sc=False),
    scratch_types=[
        pltpu.VMEM((ROWS_PER_W,), jnp.int32),
        pltpu.VMEM((ROWS_PER_W,), jnp.int32),
        pltpu.VMEM((NBUF, CHUNK, EMB), jnp.float32),
        pltpu.VMEM_SHARED((MAX_LEN, EMB), jnp.float32),
        pltpu.SemaphoreType.DMA,
        pltpu.SemaphoreType.DMA,
        pltpu.SemaphoreType.DMA,
    ],
)
def _emb_lookup(widx_hbm, pidx_hbm, word_hbm, pos_hbm, out_hbm,
                widx_v, pidx_v, rows_v, pos_sh, wsem, asem, osem):
    sid = lax.axis_index("s")
    wid = sid * NC + lax.axis_index("c")
    out_base = wid * ROWS_PER_W

    icp1 = pltpu.async_copy(widx_hbm.at[pl.ds(out_base, ROWS_PER_W)], widx_v, wsem)
    icp2 = pltpu.async_copy(pidx_hbm.at[pl.ds(out_base, ROWS_PER_W)], pidx_v, wsem)

    @pl.when(sid == 0)
    def _stage_pos_table():
        pltpu.sync_copy(pos_hbm, pos_sh)

    icp1.wait()
    icp2.wait()
    plsc.subcore_barrier()

    def word_cp(j, b):
        return pltpu.async_copy(
            word_hbm.at[widx_v.at[pl.ds(j * CHUNK, CHUNK)]], rows_v.at[b], wsem)

    def pos_cp(j, b):
        return pltpu.async_copy(
            pos_sh.at[pidx_v.at[pl.ds(j * CHUNK, CHUNK)]], rows_v.at[b], asem,
            add=True)

    def out_cp(j, b):
        return pltpu.make_async_copy(
            rows_v.at[b], out_hbm.at[pl.ds(out_base + j * CHUNK, CHUNK)], osem)

    def body(i, carry):
        j0 = i * NBUF

        @pl.when(i > 0)
        def _reclaim_buffers():
            for b in range(NBUF):
                out_cp(j0 - NBUF + b, b).wait()

        wcps = [word_cp(j0 + b, b) for b in range(NBUF)]
        acps = []
        for b in range(NBUF):
            wcps[b].wait()
            acps.append(pos_cp(j0 + b, b))
        for b in range(NBUF):
            acps[b].wait()
            out_cp(j0 + b, b).start()
        return carry

    lax.fori_loop(0, N_ITER, body, 0)
    for b in range(NBUF):
        out_cp((N_ITER - 1) * NBUF + b, b).wait()


def kernel(input_data, pos_data, word_table, pos_table):
    widx = input_data.reshape(N_IDX).astype(jnp.int32)
    pidx = pos_data.reshape(N_IDX).astype(jnp.int32)
    out = _emb_lookup(widx, pidx, word_table, pos_table)
    out = out.reshape(B, L, EMB)
    return _layout.with_layout_constraint(
        out, _layout.Layout(major_to_minor=(0, 1, 2)))
